# baseline (speedup 1.0000x reference)
import functools

import jax
import jax.numpy as jnp
from jax import lax
from jax.experimental import pallas as pl
from jax.experimental.pallas import tpu as pltpu
from jax.experimental.pallas import tpu_sc as plsc

VOCAB = 100000
D = 1024
B = 4
S = 2048
TOT = B * S

NC = 2
NS = 16
NW = NC * NS
BPW = TOT // NW
C = 32
NCHUNK = BPW // C
LANES = 16
VPR = D // LANES


def _body(tok_hbm, idx_hbm, pos_hbm, out_hbm, idx_v, tok_v, pos_v, sem):
    wid = lax.axis_index("s") * NC + lax.axis_index("c")
    base = wid * BPW
    pos_base = lax.rem(base, S)

    pltpu.sync_copy(idx_hbm.at[pl.ds(base, BPW)], idx_v)

    def chunk(c, carry):
        r0 = c * C
        pltpu.async_copy(tok_hbm.at[idx_v.at[pl.ds(r0, C)]], tok_v, sem).wait()
        pltpu.sync_copy(pos_hbm.at[pl.ds(pos_base + r0, C)], pos_v)

        def add_row(r, carry2):
            for j in range(VPR):
                sl = pl.ds(j * LANES, LANES)
                tok_v[r, sl] = tok_v[r, sl] + pos_v[r, sl]
            return carry2

        lax.fori_loop(0, C, add_row, 0, unroll=False)
        pltpu.sync_copy(tok_v, out_hbm.at[pl.ds(base + r0, C)])
        return carry

    lax.fori_loop(0, NCHUNK, chunk, 0, unroll=False)


@functools.partial(jax.jit, static_argnums=())
def _emb(tok_table, idx, pos_table):
    mesh = plsc.VectorSubcoreMesh(core_axis_name="c", subcore_axis_name="s")
    return pl.kernel(
        _body,
        out_type=jax.ShapeDtypeStruct((TOT, D), jnp.float32),
        mesh=mesh,
        scratch_types=[
            pltpu.VMEM((BPW,), jnp.int32),
            pltpu.VMEM((C, D), jnp.float32),
            pltpu.VMEM((C, D), jnp.float32),
            pltpu.SemaphoreType.DMA,
        ],
    )(tok_table, idx, pos_table)


def kernel(x, tok_table, pos_table):
    idx = x.reshape(TOT).astype(jnp.int32)
    out = _emb(tok_table, idx, pos_table)
    return out.reshape(B, S, D)


# speedup vs baseline: 1.5060x; 1.5060x over previous
import functools

import jax
import jax.numpy as jnp
from jax import lax
from jax.experimental import pallas as pl
from jax.experimental.pallas import tpu as pltpu
from jax.experimental.pallas import tpu_sc as plsc

VOCAB = 100000
D = 1024
B = 4
S = 2048
TOT = B * S

NC = 2
NS = 16
NW = NC * NS
PPW = S // NW
CC = 16
NSUB = PPW // CC
NCHUNK = NSUB * B
LANES = 16
VPR = D // LANES


def _body(tok_hbm, idx_hbm, pos_hbm, out_hbm,
          idx_v, p0, p1, t0, t1, o0, o1,
          sp0, sp1, si0, si1, so0, so1):
    w = lax.axis_index("s") * NC + lax.axis_index("c")
    pbase = w * PPW

    pbufs = (p0, p1)
    tbufs = (t0, t1)
    obufs = (o0, o1)
    psems = (sp0, sp1)
    isems = (si0, si1)
    osems = (so0, so1)

    for b in range(B):
        pltpu.sync_copy(idx_hbm.at[pl.ds(b * S + pbase, PPW)],
                        idx_v.at[pl.ds(b * PPW, PPW)])

    def start_pos(sub):
        pltpu.async_copy(pos_hbm.at[pl.ds(pbase + sub * CC, CC)],
                         pbufs[sub % 2], psems[sub % 2])

    def start_gather(c):
        sub, b = divmod(c, B)
        k = c % 2
        off = b * PPW + sub * CC
        pltpu.async_copy(tok_hbm.at[idx_v.at[pl.ds(off, CC)]],
                         tbufs[k], isems[k])

    def start_flush(c):
        sub, b = divmod(c, B)
        k = c % 2
        rbase = b * S + pbase + sub * CC
        pltpu.async_copy(obufs[k], out_hbm.at[pl.ds(rbase, CC)], osems[k])

    def wait_gather(k):
        pltpu.make_async_copy(tok_hbm.at[pl.ds(0, CC)], tbufs[k],
                              isems[k]).wait()

    def wait_pos(pb):
        pltpu.make_async_copy(pos_hbm.at[pl.ds(0, CC)], pbufs[pb],
                              psems[pb]).wait()

    def wait_flush(k):
        pltpu.make_async_copy(obufs[k], out_hbm.at[pl.ds(0, CC)],
                              osems[k]).wait()

    start_pos(0)
    start_pos(1)
    start_gather(0)
    start_gather(1)

    for c in range(NCHUNK):
        sub, b = divmod(c, B)
        k = c % 2
        pb = sub % 2
        wait_gather(k)
        if b == 0:
            wait_pos(sub % 2)
        if c >= 2:
            wait_flush(k)

        def add_row(r, carry, _k=k, _pb=pb):
            for j in range(VPR):
                sl = pl.ds(j * LANES, LANES)
                obufs[_k][r, sl] = tbufs[_k][r, sl] + pbufs[_pb][r, sl]
            return carry

        lax.fori_loop(0, CC, add_row, 0, unroll=False)

        start_flush(c)
        if c + 2 < NCHUNK:
            start_gather(c + 2)
        if b == B - 1 and sub + 2 < NSUB:
            start_pos(sub + 2)

    wait_flush(0)
    wait_flush(1)


@jax.jit
def _emb(tok_table, idx, pos_table):
    mesh = plsc.VectorSubcoreMesh(core_axis_name="c", subcore_axis_name="s")
    return pl.kernel(
        _body,
        out_type=jax.ShapeDtypeStruct((TOT, D), jnp.float32),
        mesh=mesh,
        scratch_types=[
            pltpu.VMEM((B * PPW,), jnp.int32),
            pltpu.VMEM((CC, D), jnp.float32),
            pltpu.VMEM((CC, D), jnp.float32),
            pltpu.VMEM((CC, D), jnp.float32),
            pltpu.VMEM((CC, D), jnp.float32),
            pltpu.VMEM((CC, D), jnp.float32),
            pltpu.VMEM((CC, D), jnp.float32),
            pltpu.SemaphoreType.DMA,
            pltpu.SemaphoreType.DMA,
            pltpu.SemaphoreType.DMA,
            pltpu.SemaphoreType.DMA,
            pltpu.SemaphoreType.DMA,
            pltpu.SemaphoreType.DMA,
        ],
    )(tok_table, idx, pos_table)


def kernel(x, tok_table, pos_table):
    idx = x.reshape(TOT).astype(jnp.int32)
    out = _emb(tok_table, idx, pos_table)
    return out.reshape(B, S, D)


# speedup vs baseline: 1.5272x; 1.0141x over previous
import jax
import jax.numpy as jnp
from jax import lax
from jax.experimental import pallas as pl
from jax.experimental.pallas import tpu as pltpu
from jax.experimental.pallas import tpu_sc as plsc

VOCAB = 100000
D = 1024
B = 4
S = 2048
TOT = B * S

NC = 2
NS = 16
NW = NC * NS
PPW = S // NW
CC = 16
NSUB = PPW // CC
NCHUNK = NSUB * B
NB = 5
GAHEAD = 3
LANES = 16
VPR = D // LANES


def _body(tok_hbm, idx_hbm, pos_hbm, out_hbm,
          idx_v, p0, p1, t0, t1, t2, t3, t4,
          sp0, sp1, si0, si1, si2, si3, si4, so0, so1, so2, so3, so4):
    w = lax.axis_index("s") * NC + lax.axis_index("c")
    pbase = w * PPW

    pbufs = (p0, p1)
    tbufs = (t0, t1, t2, t3, t4)
    psems = (sp0, sp1)
    isems = (si0, si1, si2, si3, si4)
    osems = (so0, so1, so2, so3, so4)

    for b in range(B):
        pltpu.sync_copy(idx_hbm.at[pl.ds(b * S + pbase, PPW)],
                        idx_v.at[pl.ds(b * PPW, PPW)])

    def start_pos(sub):
        pltpu.async_copy(pos_hbm.at[pl.ds(pbase + sub * CC, CC)],
                         pbufs[sub % 2], psems[sub % 2])

    def wait_pos(sub):
        pltpu.make_async_copy(pos_hbm.at[pl.ds(0, CC)], pbufs[sub % 2],
                              psems[sub % 2]).wait()

    def start_gather(c):
        sub, b = divmod(c, B)
        off = b * PPW + sub * CC
        pltpu.async_copy(tok_hbm.at[idx_v.at[pl.ds(off, CC)]],
                         tbufs[c % NB], isems[c % NB])

    def wait_gather(c):
        pltpu.make_async_copy(tok_hbm.at[pl.ds(0, CC)], tbufs[c % NB],
                              isems[c % NB]).wait()

    def start_flush(c):
        sub, b = divmod(c, B)
        rbase = b * S + pbase + sub * CC
        pltpu.async_copy(tbufs[c % NB], out_hbm.at[pl.ds(rbase, CC)],
                         osems[c % NB])

    def wait_flush(c):
        pltpu.make_async_copy(tbufs[c % NB], out_hbm.at[pl.ds(0, CC)],
                              osems[c % NB]).wait()

    start_pos(0)
    start_pos(1)
    for c in range(GAHEAD):
        start_gather(c)

    for c in range(NCHUNK):
        sub, b = divmod(c, B)
        if c + GAHEAD < NCHUNK:
            if c + GAHEAD >= NB:
                wait_flush(c + GAHEAD - NB)
            start_gather(c + GAHEAD)
        wait_gather(c)
        if b == 0:
            wait_pos(sub)

        def add_row(r, carry, _k=c % NB, _pb=sub % 2):
            for j in range(VPR):
                sl = pl.ds(j * LANES, LANES)
                tbufs[_k][r, sl] = tbufs[_k][r, sl] + pbufs[_pb][r, sl]
            return carry

        lax.fori_loop(0, CC, add_row, 0, unroll=False)

        if b == B - 1 and sub + 2 < NSUB:
            start_pos(sub + 2)
        start_flush(c)

    for c in range(NCHUNK - NB, NCHUNK):
        wait_flush(c)


@jax.jit
def _emb(tok_table, idx, pos_table):
    mesh = plsc.VectorSubcoreMesh(core_axis_name="c", subcore_axis_name="s")
    return pl.kernel(
        _body,
        out_type=jax.ShapeDtypeStruct((TOT, D), jnp.float32),
        mesh=mesh,
        scratch_types=[
            pltpu.VMEM((B * PPW,), jnp.int32),
            pltpu.VMEM((CC, D), jnp.float32),
            pltpu.VMEM((CC, D), jnp.float32),
            pltpu.VMEM((CC, D), jnp.float32),
            pltpu.VMEM((CC, D), jnp.float32),
            pltpu.VMEM((CC, D), jnp.float32),
            pltpu.VMEM((CC, D), jnp.float32),
            pltpu.VMEM((CC, D), jnp.float32),
            pltpu.SemaphoreType.DMA,
            pltpu.SemaphoreType.DMA,
            pltpu.SemaphoreType.DMA,
            pltpu.SemaphoreType.DMA,
            pltpu.SemaphoreType.DMA,
            pltpu.SemaphoreType.DMA,
            pltpu.SemaphoreType.DMA,
            pltpu.SemaphoreType.DMA,
            pltpu.SemaphoreType.DMA,
            pltpu.SemaphoreType.DMA,
            pltpu.SemaphoreType.DMA,
            pltpu.SemaphoreType.DMA,
        ],
    )(tok_table, idx, pos_table)


def kernel(x, tok_table, pos_table):
    idx = x.reshape(TOT).astype(jnp.int32)
    out = _emb(tok_table, idx, pos_table)
    return out.reshape(B, S, D)
